# Initial kernel scaffold; baseline (speedup 1.0000x reference)
import jax
import jax.numpy as jnp
from jax.experimental import pallas as pl


def kernel(X, grid_part, grid_part_norm, int_map):
    raise NotImplementedError("write your pallas kernel here")


# speedup vs baseline: 2.9896x; 2.9896x over previous
import jax
import jax.numpy as jnp
from jax.experimental import pallas as pl

_N_CODES = 227
_CODESZ = 8
_BLOCK = 4096


def _quant_kernel(x_ref, gpt2_ref, gpn_ref, gp_ref, vals_ref, idx_ref):
    x = x_ref[...]
    neg = x < 0.0
    sign = jnp.where(neg, -1.0, 1.0)
    xa = jnp.abs(x)

    t = jax.lax.dot_general(
        xa.astype(jnp.bfloat16), gpt2_ref[...].astype(jnp.bfloat16),
        (((1,), (0,)), ((), ())),
        preferred_element_type=jnp.float32,
    )
    scores = t - gpn_ref[...]

    m = jnp.max(scores, axis=1, keepdims=True)
    iota = jax.lax.broadcasted_iota(jnp.int32, scores.shape, 1)
    qidx = jnp.min(
        jnp.where(scores == m, iota, _N_CODES), axis=1, keepdims=True
    )

    onehot = (iota == qidx).astype(jnp.float32)
    vals_abs = jax.lax.dot_general(
        onehot, gp_ref[...], (((1,), (0,)), ((), ())),
        preferred_element_type=jnp.float32,
    )
    vals_ref[...] = vals_abs * sign

    k_iota = jax.lax.broadcasted_iota(jnp.int32, x.shape, 1)
    flips = jnp.sum(
        jnp.where(neg, jnp.left_shift(1, k_iota), 0), axis=1, keepdims=True
    )
    idx_ref[...] = (flips << 8) + qidx - 32768


def kernel(X, grid_part, grid_part_norm, int_map):
    del int_map
    n = X.shape[0]
    b = _BLOCK
    gpt2 = (2.0 * grid_part).T
    gpn = grid_part_norm[None, :]

    vals, idx32 = pl.pallas_call(
        _quant_kernel,
        grid=(n // b,),
        in_specs=[
            pl.BlockSpec((b, _CODESZ), lambda i: (i, 0)),
            pl.BlockSpec((_CODESZ, _N_CODES), lambda i: (0, 0)),
            pl.BlockSpec((1, _N_CODES), lambda i: (0, 0)),
            pl.BlockSpec((_N_CODES, _CODESZ), lambda i: (0, 0)),
        ],
        out_specs=[
            pl.BlockSpec((b, _CODESZ), lambda i: (i, 0)),
            pl.BlockSpec((b, 1), lambda i: (i, 0)),
        ],
        out_shape=[
            jax.ShapeDtypeStruct((n, _CODESZ), jnp.float32),
            jax.ShapeDtypeStruct((n, 1), jnp.int32),
        ],
    )(X, gpt2, gpn, grid_part)

    return vals, idx32[:, 0].astype(jnp.int16)
